# baseline (speedup 1.0000x reference)
import functools

import jax
import jax.numpy as jnp
from jax import lax
from jax.experimental import pallas as pl
from jax.experimental.pallas import tpu as pltpu
from jax.experimental.pallas import tpu_sc as plsc

B = 64
L_IN = 1536
D_TISSUE = 64
D_TOKEN = 32
HIDDEN = 1024
H2 = HIDDEN // 2
K_SEQ = L_IN * D_TOKEN

NC = 2
NS = 16
NW = NC * NS

KB = 2048
SEGS = (2, 6, 8, 8)

_SQRT_HALF = 0.7071067811865476


def _gelu(x):
    return 0.5 * x * (1.0 + lax.erf(x * _SQRT_HALF))


def _sc_gather(ntok, with_tissue):
    tpw = ntok // NW
    mesh = plsc.VectorSubcoreMesh(core_axis_name="c", subcore_axis_name="s")
    out_type = [jax.ShapeDtypeStruct((ntok, D_TOKEN), jnp.float32)]
    scratch = [
        pltpu.VMEM((tpw,), jnp.int32),
        pltpu.VMEM((tpw, D_TOKEN), jnp.float32),
        pltpu.SemaphoreType.DMA,
    ]
    if with_tissue:
        out_type.append(jax.ShapeDtypeStruct((B, D_TISSUE), jnp.float32))
        scratch += [pltpu.VMEM((B,), jnp.int32),
                    pltpu.VMEM((B, D_TISSUE), jnp.float32)]

    @functools.partial(
        pl.kernel,
        out_type=tuple(out_type),
        mesh=mesh,
        scratch_types=tuple(scratch),
        compiler_params=pltpu.CompilerParams(use_tc_tiling_on_sc=False),
    )
    def body(*refs):
        it = iter(refs)
        seq_hbm, idx_hbm = next(it), next(it)
        if with_tissue:
            ttab_hbm, tid_hbm = next(it), next(it)
        x_hbm = next(it)
        if with_tissue:
            te_hbm = next(it)
        idxv, rowsv, sem = next(it), next(it), next(it)
        if with_tissue:
            tidv, trowsv = next(it), next(it)
        wid = lax.axis_index("s") * NC + lax.axis_index("c")
        pltpu.sync_copy(idx_hbm.at[pl.ds(wid * tpw, tpw)], idxv)
        pltpu.async_copy(seq_hbm.at[idxv], rowsv, sem).wait()
        pltpu.sync_copy(rowsv, x_hbm.at[pl.ds(wid * tpw, tpw)])

        if with_tissue:
            @pl.when(wid == 0)
            def _():
                pltpu.sync_copy(tid_hbm, tidv)
                pltpu.async_copy(ttab_hbm.at[tidv], trowsv, sem).wait()
                pltpu.sync_copy(trowsv, te_hbm)

    return body


def _mlp_seg(x2d, W1, kstart, head, tail):
    nk = x2d.shape[1] // KB
    row0 = D_TISSUE + kstart * KB
    first = len(head) == 2
    last = tail is not None

    def body(*refs):
        it = iter(refs)
        x_ref = next(it)
        if first:
            te_ref, b1_ref = next(it), next(it)
        else:
            accin_ref = next(it)
        if last:
            w2_ref, b2_ref, w3_ref, b3_ref = next(it), next(it), next(it), next(it)
        w1_hbm = next(it)
        out_ref = next(it)
        w1buf, accv, sems = next(it), next(it), next(it)
        if first:
            w1t, semt = next(it), next(it)
        k = pl.program_id(0)

        def w1_copy(kk, slot):
            return pltpu.make_async_copy(
                w1_hbm.at[pl.ds(row0 + kk * KB, KB), :],
                w1buf.at[slot], sems.at[slot])

        @pl.when(k == 0)
        def _():
            if first:
                pltpu.make_async_copy(
                    w1_hbm.at[pl.ds(0, D_TISSUE), :], w1t, semt).start()
            w1_copy(0, 0).start()

        @pl.when(k + 1 < nk)
        def _():
            w1_copy(k + 1, (k + 1) % 2).start()

        @pl.when(k == 0)
        def _():
            if first:
                pltpu.make_async_copy(
                    w1_hbm.at[pl.ds(0, D_TISSUE), :], w1t, semt).wait()
                accv[...] = (
                    jnp.dot(te_ref[...], w1t[...],
                            preferred_element_type=jnp.float32) + b1_ref[...])
            else:
                accv[...] = accin_ref[...]

        w1_copy(k, k % 2).wait()
        accv[...] += jnp.dot(x_ref[...], w1buf[k % 2],
                             preferred_element_type=jnp.float32)

        @pl.when(k == nk - 1)
        def _():
            if last:
                h = _gelu(accv[...])
                h2 = _gelu(jnp.dot(h, w2_ref[...],
                                   preferred_element_type=jnp.float32)
                           + b2_ref[...])
                out_ref[...] = (
                    jnp.dot(h2, w3_ref[...], preferred_element_type=jnp.float32)
                    + b3_ref[...])
            else:
                out_ref[...] = accv[...]

    in_specs = [pl.BlockSpec((B, KB), lambda k: (0, k))]
    if first:
        in_specs += [pl.BlockSpec((B, D_TISSUE), lambda k: (0, 0)),
                     pl.BlockSpec((1, HIDDEN), lambda k: (0, 0))]
    else:
        in_specs += [pl.BlockSpec((B, HIDDEN), lambda k: (0, 0))]
    if last:
        in_specs += [pl.BlockSpec((HIDDEN, H2), lambda k: (0, 0)),
                     pl.BlockSpec((1, H2), lambda k: (0, 0)),
                     pl.BlockSpec((H2, 1), lambda k: (0, 0)),
                     pl.BlockSpec((1, 1), lambda k: (0, 0))]
    in_specs += [pl.BlockSpec(memory_space=pl.ANY)]
    out_shape = (jax.ShapeDtypeStruct((B, 1), jnp.float32) if last
                 else jax.ShapeDtypeStruct((B, HIDDEN), jnp.float32))
    out_spec = (pl.BlockSpec((B, 1), lambda k: (0, 0)) if last
                else pl.BlockSpec((B, HIDDEN), lambda k: (0, 0)))
    scratch = [
        pltpu.VMEM((2, KB, HIDDEN), jnp.float32),
        pltpu.VMEM((B, HIDDEN), jnp.float32),
        pltpu.SemaphoreType.DMA((2,)),
    ]
    if first:
        scratch += [pltpu.VMEM((D_TISSUE, HIDDEN), jnp.float32),
                    pltpu.SemaphoreType.DMA]

    return pl.pallas_call(
        body,
        grid=(nk,),
        in_specs=in_specs,
        out_specs=out_spec,
        out_shape=out_shape,
        scratch_shapes=scratch,
        compiler_params=pltpu.CompilerParams(
            dimension_semantics=("arbitrary",)),
    )(x2d, *head, *(tail or ()), W1)


def kernel(rna_data, tissue_id, tissue_table, seq_table, W1, b1, W2, b2, W3, b3):
    b1r = b1.reshape(1, HIDDEN)
    tail = (W2, b2.reshape(1, H2), W3, b3.reshape(1, 1))
    nseg = len(SEGS)
    pos = 0
    xs, te = [], None
    def emit_sc(i, pos):
        npos = SEGS[i] * KB // D_TOKEN
        ntok = B * npos
        idx = rna_data[:, pos:pos + npos].reshape(ntok)
        if i == 0:
            x3, te_ = _sc_gather(ntok, True)(seq_table, idx, tissue_table,
                                             tissue_id)
        else:
            (x3,) = _sc_gather(ntok, False)(seq_table, idx)
            te_ = None
        return x3.reshape(B, SEGS[i] * KB), te_, pos + npos

    x0, te, pos = emit_sc(0, pos)
    xs.append(x0)
    acc = None
    kstart = 0
    for i in range(nseg):
        if i + 1 < nseg:
            xn, _, pos = emit_sc(i + 1, pos)
            xs.append(xn)
        head = (te, b1r) if i == 0 else (acc,)
        seg_tail = tail if i == nseg - 1 else None
        acc = _mlp_seg(xs[i], W1, kstart, head, seg_tail)
        kstart += SEGS[i]
    return acc


# speedup vs baseline: 1.4156x; 1.4156x over previous
import functools

import jax
import jax.numpy as jnp
from jax import lax
from jax.experimental import pallas as pl
from jax.experimental.pallas import tpu as pltpu
from jax.experimental.pallas import tpu_sc as plsc

B = 64
L_IN = 1536
D_TISSUE = 64
D_TOKEN = 32
HIDDEN = 1024
H2 = HIDDEN // 2
K_SEQ = L_IN * D_TOKEN
VOCAB = 65

D_PAIR = 2 * D_TOKEN
NPAIR = B * L_IN // 2

NC = 2
NS = 16
NW = NC * NS
PPW = NPAIR // NW

KB = 2048
NK = K_SEQ // KB

_SQRT_HALF = 0.7071067811865476


def _gelu(x):
    return 0.5 * x * (1.0 + lax.erf(x * _SQRT_HALF))


def _sc_gather(pair_table, pair_idx, tissue_table, tissue_id):
    mesh = plsc.VectorSubcoreMesh(core_axis_name="c", subcore_axis_name="s")

    @functools.partial(
        pl.kernel,
        out_type=(
            jax.ShapeDtypeStruct((NPAIR, D_PAIR), jnp.float32),
            jax.ShapeDtypeStruct((B, D_TISSUE), jnp.float32),
        ),
        mesh=mesh,
        scratch_types=(
            pltpu.VMEM((PPW,), jnp.int32),
            pltpu.VMEM((PPW, D_PAIR), jnp.float32),
            pltpu.VMEM((B,), jnp.int32),
            pltpu.VMEM((B, D_TISSUE), jnp.float32),
            pltpu.SemaphoreType.DMA,
        ),
        compiler_params=pltpu.CompilerParams(use_tc_tiling_on_sc=False),
    )
    def body(ptab_hbm, idx_hbm, ttab_hbm, tid_hbm, x_hbm, te_hbm,
             idxv, rowsv, tidv, trowsv, sem):
        wid = lax.axis_index("s") * NC + lax.axis_index("c")
        pltpu.sync_copy(idx_hbm.at[pl.ds(wid * PPW, PPW)], idxv)
        pltpu.async_copy(ptab_hbm.at[idxv], rowsv, sem).wait()
        pltpu.sync_copy(rowsv, x_hbm.at[pl.ds(wid * PPW, PPW)])

        @pl.when(wid == 0)
        def _():
            pltpu.sync_copy(tid_hbm, tidv)
            pltpu.async_copy(ttab_hbm.at[tidv], trowsv, sem).wait()
            pltpu.sync_copy(trowsv, te_hbm)

    return body(pair_table, pair_idx, tissue_table, tissue_id)


def _mlp(x2d, te, b1r, W2, b2r, W3, b3r, W1):
    def body(x_ref, te_ref, b1_ref, w2_ref, b2_ref, w3_ref, b3_ref, w1_hbm,
             y_ref, w1buf, w1t, acc, sems, semt):
        k = pl.program_id(0)

        def w1_copy(kk, slot):
            return pltpu.make_async_copy(
                w1_hbm.at[pl.ds(D_TISSUE + kk * KB, KB), :],
                w1buf.at[slot], sems.at[slot])

        @pl.when(k == 0)
        def _():
            pltpu.make_async_copy(w1_hbm.at[pl.ds(0, D_TISSUE), :], w1t, semt).start()
            w1_copy(0, 0).start()

        @pl.when(k + 1 < NK)
        def _():
            w1_copy(k + 1, (k + 1) % 2).start()

        @pl.when(k == 0)
        def _():
            pltpu.make_async_copy(w1_hbm.at[pl.ds(0, D_TISSUE), :], w1t, semt).wait()
            acc[...] = (
                jnp.dot(te_ref[...], w1t[...], preferred_element_type=jnp.float32)
                + b1_ref[...])

        w1_copy(k, k % 2).wait()
        acc[...] += jnp.dot(x_ref[...], w1buf[k % 2],
                            preferred_element_type=jnp.float32)

        @pl.when(k == NK - 1)
        def _():
            h = _gelu(acc[...])
            h2 = _gelu(jnp.dot(h, w2_ref[...], preferred_element_type=jnp.float32)
                       + b2_ref[...])
            y_ref[...] = (
                jnp.dot(h2, w3_ref[...], preferred_element_type=jnp.float32)
                + b3_ref[...])

    return pl.pallas_call(
        body,
        grid=(NK,),
        in_specs=[
            pl.BlockSpec((B, KB), lambda k: (0, k)),
            pl.BlockSpec((B, D_TISSUE), lambda k: (0, 0)),
            pl.BlockSpec((1, HIDDEN), lambda k: (0, 0)),
            pl.BlockSpec((HIDDEN, H2), lambda k: (0, 0)),
            pl.BlockSpec((1, H2), lambda k: (0, 0)),
            pl.BlockSpec((H2, 1), lambda k: (0, 0)),
            pl.BlockSpec((1, 1), lambda k: (0, 0)),
            pl.BlockSpec(memory_space=pl.ANY),
        ],
        out_specs=pl.BlockSpec((B, 1), lambda k: (0, 0)),
        out_shape=jax.ShapeDtypeStruct((B, 1), jnp.float32),
        scratch_shapes=[
            pltpu.VMEM((2, KB, HIDDEN), jnp.float32),
            pltpu.VMEM((D_TISSUE, HIDDEN), jnp.float32),
            pltpu.VMEM((B, HIDDEN), jnp.float32),
            pltpu.SemaphoreType.DMA((2,)),
            pltpu.SemaphoreType.DMA,
        ],
        compiler_params=pltpu.CompilerParams(
            dimension_semantics=("arbitrary",)),
    )(x2d, te, b1r, W2, b2r, W3, b3r, W1)


def kernel(rna_data, tissue_id, tissue_table, seq_table, W1, b1, W2, b2, W3, b3):
    pair_table = jnp.concatenate(
        [jnp.repeat(seq_table, VOCAB, axis=0),
         jnp.tile(seq_table, (VOCAB, 1))], axis=1)
    pair_idx = (rna_data[:, 0::2] * VOCAB + rna_data[:, 1::2]).reshape(NPAIR)
    x3, te = _sc_gather(pair_table, pair_idx, tissue_table, tissue_id)
    x2d = x3.reshape(B, K_SEQ)
    return _mlp(x2d, te, b1.reshape(1, HIDDEN), W2, b2.reshape(1, H2),
                W3, b3.reshape(1, 1), W1)


# speedup vs baseline: 1.5942x; 1.1262x over previous
import functools

import jax
import jax.numpy as jnp
from jax import lax
from jax.experimental import pallas as pl
from jax.experimental.pallas import tpu as pltpu
from jax.experimental.pallas import tpu_sc as plsc

B = 64
L_IN = 1536
D_TISSUE = 64
D_TOKEN = 32
HIDDEN = 1024
H2 = HIDDEN // 2
K_SEQ = L_IN * D_TOKEN
VOCAB = 65

D_PAIR = 2 * D_TOKEN
NPAIR = B * L_IN // 2

NC = 2
NS = 16
NW = NC * NS
PPW = NPAIR // NW

KB = 2048
NK = K_SEQ // KB

_SQRT_HALF = 0.7071067811865476


def _gelu(x):
    return 0.5 * x * (1.0 + lax.erf(x * _SQRT_HALF))


def _sc_gather(pair_table, tok_flat, tissue_table, tissue_id):
    mesh = plsc.VectorSubcoreMesh(core_axis_name="c", subcore_axis_name="s")
    tpw = 2 * PPW

    @functools.partial(
        pl.kernel,
        out_type=(
            jax.ShapeDtypeStruct((NPAIR, D_PAIR), jnp.float32),
            jax.ShapeDtypeStruct((B, D_TISSUE), jnp.float32),
        ),
        mesh=mesh,
        scratch_types=(
            pltpu.VMEM((tpw,), jnp.int32),
            pltpu.VMEM((PPW,), jnp.int32),
            pltpu.VMEM((PPW, D_PAIR), jnp.float32),
            pltpu.VMEM((B,), jnp.int32),
            pltpu.VMEM((B, D_TISSUE), jnp.float32),
            pltpu.SemaphoreType.DMA,
        ),
        compiler_params=pltpu.CompilerParams(use_tc_tiling_on_sc=False,
                                             needs_layout_passes=False),
    )
    def body(ptab_hbm, tok_hbm, ttab_hbm, tid_hbm, x_hbm, te_hbm,
             tokv, idxv, rowsv, tidv, trowsv, sem):
        wid = lax.axis_index("s") * NC + lax.axis_index("c")
        lane = lax.iota(jnp.int32, 16)
        pltpu.sync_copy(tok_hbm.at[pl.ds(wid * tpw, tpw)], tokv)

        def chunk(c, carry):
            basev = c * 32 + 2 * lane
            ev = plsc.load_gather(tokv, [basev])
            ov = plsc.load_gather(tokv, [basev + 1])
            idxv[pl.ds(c * 16, 16)] = ev * VOCAB + ov
            return carry

        lax.fori_loop(0, PPW // 16, chunk, 0)
        pltpu.async_copy(ptab_hbm.at[idxv], rowsv, sem).wait()
        pltpu.sync_copy(rowsv, x_hbm.at[pl.ds(wid * PPW, PPW)])

        @pl.when(wid == 0)
        def _():
            pltpu.sync_copy(tid_hbm, tidv)
            pltpu.async_copy(ttab_hbm.at[tidv], trowsv, sem).wait()
            pltpu.sync_copy(trowsv, te_hbm)

    return body(pair_table, tok_flat, tissue_table, tissue_id)


def _mlp(x2d, te, b1r, W2, b2r, W3, b3r, W1):
    def body(x_ref, te_ref, b1_ref, w2_ref, b2_ref, w3_ref, b3_ref, w1_hbm,
             y_ref, w1buf, w1t, acc, sems, semt):
        k = pl.program_id(0)

        def w1_copy(kk, slot):
            return pltpu.make_async_copy(
                w1_hbm.at[pl.ds(D_TISSUE + kk * KB, KB), :],
                w1buf.at[slot], sems.at[slot])

        @pl.when(k == 0)
        def _():
            pltpu.make_async_copy(w1_hbm.at[pl.ds(0, D_TISSUE), :], w1t, semt).start()
            w1_copy(0, 0).start()

        @pl.when(k + 1 < NK)
        def _():
            w1_copy(k + 1, (k + 1) % 2).start()

        @pl.when(k == 0)
        def _():
            pltpu.make_async_copy(w1_hbm.at[pl.ds(0, D_TISSUE), :], w1t, semt).wait()
            acc[...] = (
                jnp.dot(te_ref[...], w1t[...], preferred_element_type=jnp.float32)
                + b1_ref[...])

        w1_copy(k, k % 2).wait()
        acc[...] += jnp.dot(x_ref[...], w1buf[k % 2],
                            preferred_element_type=jnp.float32)

        @pl.when(k == NK - 1)
        def _():
            h = _gelu(acc[...])
            h2 = _gelu(jnp.dot(h, w2_ref[...], preferred_element_type=jnp.float32)
                       + b2_ref[...])
            y_ref[...] = (
                jnp.dot(h2, w3_ref[...], preferred_element_type=jnp.float32)
                + b3_ref[...])

    return pl.pallas_call(
        body,
        grid=(NK,),
        in_specs=[
            pl.BlockSpec((B, KB), lambda k: (0, k)),
            pl.BlockSpec((B, D_TISSUE), lambda k: (0, 0)),
            pl.BlockSpec((1, HIDDEN), lambda k: (0, 0)),
            pl.BlockSpec((HIDDEN, H2), lambda k: (0, 0)),
            pl.BlockSpec((1, H2), lambda k: (0, 0)),
            pl.BlockSpec((H2, 1), lambda k: (0, 0)),
            pl.BlockSpec((1, 1), lambda k: (0, 0)),
            pl.BlockSpec(memory_space=pl.ANY),
        ],
        out_specs=pl.BlockSpec((B, 1), lambda k: (0, 0)),
        out_shape=jax.ShapeDtypeStruct((B, 1), jnp.float32),
        scratch_shapes=[
            pltpu.VMEM((2, KB, HIDDEN), jnp.float32),
            pltpu.VMEM((D_TISSUE, HIDDEN), jnp.float32),
            pltpu.VMEM((B, HIDDEN), jnp.float32),
            pltpu.SemaphoreType.DMA((2,)),
            pltpu.SemaphoreType.DMA,
        ],
        compiler_params=pltpu.CompilerParams(
            dimension_semantics=("arbitrary",)),
    )(x2d, te, b1r, W2, b2r, W3, b3r, W1)


def kernel(rna_data, tissue_id, tissue_table, seq_table, W1, b1, W2, b2, W3, b3):
    pair_table = jnp.concatenate(
        [jnp.repeat(seq_table, VOCAB, axis=0),
         jnp.tile(seq_table, (VOCAB, 1))], axis=1)
    x3, te = _sc_gather(pair_table, rna_data.reshape(B * L_IN), tissue_table,
                        tissue_id)
    x2d = x3.reshape(B, K_SEQ)
    return _mlp(x2d, te, b1.reshape(1, HIDDEN), W2, b2.reshape(1, H2),
                W3, b3.reshape(1, 1), W1)


# speedup vs baseline: 1.7378x; 1.0900x over previous
import functools

import jax
import jax.numpy as jnp
from jax import lax
from jax.experimental import pallas as pl
from jax.experimental.pallas import tpu as pltpu
from jax.experimental.pallas import tpu_sc as plsc

B = 64
L_IN = 1536
D_TISSUE = 64
D_TOKEN = 32
HIDDEN = 1024
H2 = HIDDEN // 2
K_SEQ = L_IN * D_TOKEN
VOCAB = 65

D_PAIR = 2 * D_TOKEN
NPAIR = B * L_IN // 2

NC = 2
NS = 16
NW = NC * NS
PPW = NPAIR // NW

KB = 2048
NK = K_SEQ // KB

_SQRT_HALF = 0.7071067811865476


def _gelu(x):
    return 0.5 * x * (1.0 + lax.erf(x * _SQRT_HALF))


def _sc_gather(pair_table, rna2d, tissue_table, tissue_id):
    mesh = plsc.VectorSubcoreMesh(core_axis_name="c", subcore_axis_name="s")
    ppos = 2 * PPW // B

    @functools.partial(
        pl.kernel,
        out_type=(
            jax.ShapeDtypeStruct((NPAIR, D_PAIR), jnp.float32),
            jax.ShapeDtypeStruct((B, D_TISSUE), jnp.float32),
        ),
        mesh=mesh,
        scratch_types=(
            pltpu.VMEM((B, ppos), jnp.int32),
            pltpu.VMEM((PPW,), jnp.int32),
            pltpu.VMEM((PPW, D_PAIR), jnp.float32),
            pltpu.VMEM((B,), jnp.int32),
            pltpu.VMEM((B, D_TISSUE), jnp.float32),
            pltpu.SemaphoreType.DMA,
        ),
        compiler_params=pltpu.CompilerParams(use_tc_tiling_on_sc=False,
                                             needs_layout_passes=False),
    )
    def body(ptab_hbm, tok_hbm, ttab_hbm, tid_hbm, x_hbm, te_hbm,
             tokv, idxv, rowsv, tidv, trowsv, sem):
        wid = lax.axis_index("s") * NC + lax.axis_index("c")
        lane = lax.iota(jnp.int32, 16)
        pltpu.sync_copy(tok_hbm.at[:, pl.ds(ppos * wid, ppos)], tokv)

        def chunk(c, carry):
            bv = 8 * (c % 8) + lane // 2
            pv = 4 * (c // 8) + 2 * (lane % 2)
            ev = plsc.load_gather(tokv, [bv, pv])
            ov = plsc.load_gather(tokv, [bv, pv + 1])
            idxv[pl.ds(c * 16, 16)] = ev * VOCAB + ov
            return carry

        lax.fori_loop(0, PPW // 16, chunk, 0)
        pltpu.async_copy(ptab_hbm.at[idxv], rowsv, sem).wait()
        pltpu.sync_copy(rowsv, x_hbm.at[pl.ds(wid * PPW, PPW)])

        @pl.when(wid == 0)
        def _():
            pltpu.sync_copy(tid_hbm, tidv)
            pltpu.async_copy(ttab_hbm.at[tidv], trowsv, sem).wait()
            pltpu.sync_copy(trowsv, te_hbm)

    return body(pair_table, rna2d, tissue_table, tissue_id)


def _mlp(x2d, te, b1r, W2, b2r, W3, b3r, W1):
    def body(x_ref, te_ref, b1_ref, w2_ref, b2_ref, w3_ref, b3_ref, w1_hbm,
             y_ref, w1buf, w1t, acc, sems, semt):
        k = pl.program_id(0)

        def w1_copy(kk, slot):
            return pltpu.make_async_copy(
                w1_hbm.at[pl.ds(D_TISSUE + kk * KB, KB), :],
                w1buf.at[slot], sems.at[slot])

        @pl.when(k == 0)
        def _():
            pltpu.make_async_copy(w1_hbm.at[pl.ds(0, D_TISSUE), :], w1t, semt).start()
            w1_copy(0, 0).start()

        @pl.when(k + 1 < NK)
        def _():
            w1_copy(k + 1, (k + 1) % 2).start()

        @pl.when(k == 0)
        def _():
            pltpu.make_async_copy(w1_hbm.at[pl.ds(0, D_TISSUE), :], w1t, semt).wait()
            acc[...] = (
                jnp.dot(te_ref[...], w1t[...], preferred_element_type=jnp.float32)
                + b1_ref[...])

        w1_copy(k, k % 2).wait()
        part = jnp.dot(x_ref[0], w1buf[k % 2, pl.ds(0, 128), :],
                       preferred_element_type=jnp.float32)
        for qq in range(1, KB // 128):
            part += jnp.dot(x_ref[qq], w1buf[k % 2, pl.ds(128 * qq, 128), :],
                            preferred_element_type=jnp.float32)
        acc[...] += part

        @pl.when(k == NK - 1)
        def _():
            h = _gelu(acc[...])
            h2 = _gelu(jnp.dot(h, w2_ref[...], preferred_element_type=jnp.float32)
                       + b2_ref[...])
            y_ref[...] = (
                jnp.dot(h2, w3_ref[...], preferred_element_type=jnp.float32)
                + b3_ref[...])

    return pl.pallas_call(
        body,
        grid=(NK,),
        in_specs=[
            pl.BlockSpec((KB // 128, B, 128), lambda k: (k, 0, 0)),
            pl.BlockSpec((B, D_TISSUE), lambda k: (0, 0)),
            pl.BlockSpec((1, HIDDEN), lambda k: (0, 0)),
            pl.BlockSpec((HIDDEN, H2), lambda k: (0, 0)),
            pl.BlockSpec((1, H2), lambda k: (0, 0)),
            pl.BlockSpec((H2, 1), lambda k: (0, 0)),
            pl.BlockSpec((1, 1), lambda k: (0, 0)),
            pl.BlockSpec(memory_space=pl.ANY),
        ],
        out_specs=pl.BlockSpec((B, 1), lambda k: (0, 0)),
        out_shape=jax.ShapeDtypeStruct((B, 1), jnp.float32),
        scratch_shapes=[
            pltpu.VMEM((2, KB, HIDDEN), jnp.float32),
            pltpu.VMEM((D_TISSUE, HIDDEN), jnp.float32),
            pltpu.VMEM((B, HIDDEN), jnp.float32),
            pltpu.SemaphoreType.DMA((2,)),
            pltpu.SemaphoreType.DMA,
        ],
        compiler_params=pltpu.CompilerParams(
            dimension_semantics=("arbitrary",)),
    )(x2d, te, b1r, W2, b2r, W3, b3r, W1)


def kernel(rna_data, tissue_id, tissue_table, seq_table, W1, b1, W2, b2, W3, b3):
    pair_table = jnp.concatenate(
        [jnp.repeat(seq_table, VOCAB, axis=0),
         jnp.tile(seq_table, (VOCAB, 1))], axis=1)
    x3, te = _sc_gather(pair_table, rna_data, tissue_table, tissue_id)
    x6 = x3.reshape(K_SEQ // 128, B, 128)
    return _mlp(x6, te, b1.reshape(1, HIDDEN), W2, b2.reshape(1, H2),
                W3, b3.reshape(1, 1), W1)


# speedup vs baseline: 1.8009x; 1.0363x over previous
import functools

import jax
import jax.numpy as jnp
from jax import lax
from jax.experimental import pallas as pl
from jax.experimental.pallas import tpu as pltpu
from jax.experimental.pallas import tpu_sc as plsc

B = 64
L_IN = 1536
D_TISSUE = 64
D_TOKEN = 32
HIDDEN = 1024
H2 = HIDDEN // 2
K_SEQ = L_IN * D_TOKEN
VOCAB = 65

D_PAIR = 2 * D_TOKEN
NPAIR = B * L_IN // 2

NC = 2
NS = 16
NW = NC * NS
PPW = NPAIR // NW

KB = 2048
NK = K_SEQ // KB

_SQRT_HALF = 0.7071067811865476


def _gelu(x):
    return 0.5 * x * (1.0 + lax.erf(x * _SQRT_HALF))


def _sc_gather(pair_table, rna2d, tissue_table, tissue_id):
    mesh = plsc.VectorSubcoreMesh(core_axis_name="c", subcore_axis_name="s")
    ppos = 2 * PPW // B

    @functools.partial(
        pl.kernel,
        out_type=(
            jax.ShapeDtypeStruct((NPAIR, D_PAIR), jnp.float32),
            jax.ShapeDtypeStruct((B, D_TISSUE), jnp.float32),
        ),
        mesh=mesh,
        scratch_types=(
            pltpu.VMEM((B, ppos), jnp.int32),
            pltpu.VMEM((PPW,), jnp.int32),
            pltpu.VMEM((PPW, D_PAIR), jnp.float32),
            pltpu.VMEM((B,), jnp.int32),
            pltpu.VMEM((B, D_TISSUE), jnp.float32),
            pltpu.SemaphoreType.DMA,
        ),
        compiler_params=pltpu.CompilerParams(use_tc_tiling_on_sc=False,
                                             needs_layout_passes=False),
    )
    def body(ptab_hbm, tok_hbm, ttab_hbm, tid_hbm, x_hbm, te_hbm,
             tokv, idxv, rowsv, tidv, trowsv, sem):
        wid = lax.axis_index("s") * NC + lax.axis_index("c")
        lane = lax.iota(jnp.int32, 16)
        pltpu.sync_copy(tok_hbm.at[:, pl.ds(ppos * wid, ppos)], tokv)

        def chunk(c, carry):
            bv = 8 * (c % 8) + lane // 2
            pv = 4 * (c // 8) + 2 * (lane % 2)
            ev = plsc.load_gather(tokv, [bv, pv])
            ov = plsc.load_gather(tokv, [bv, pv + 1])
            idxv[pl.ds(c * 16, 16)] = ev * VOCAB + ov
            return carry

        lax.fori_loop(0, PPW // 16, chunk, 0)
        pltpu.async_copy(ptab_hbm.at[idxv], rowsv, sem).wait()
        pltpu.sync_copy(rowsv, x_hbm.at[pl.ds(wid * PPW, PPW)])

        @pl.when(wid == 0)
        def _():
            pltpu.sync_copy(tid_hbm, tidv)
            pltpu.async_copy(ttab_hbm.at[tidv], trowsv, sem).wait()
            pltpu.sync_copy(trowsv, te_hbm)

    return body(pair_table, rna2d, tissue_table, tissue_id)


def _mlp(x2d, te, b1r, W2, b2r, W3, b3r, W1):
    def body(x_ref, te_ref, b1_ref, w2_ref, b2_ref, w3_ref, b3_ref, w1_hbm,
             y_ref, w1buf, w1t, acc, sems, semt):
        k = pl.program_id(0)

        def w1_copy(kk, slot):
            return pltpu.make_async_copy(
                w1_hbm.at[pl.ds(D_TISSUE + kk * KB, KB), :],
                w1buf.at[slot], sems.at[slot])

        @pl.when(k == 0)
        def _():
            pltpu.make_async_copy(w1_hbm.at[pl.ds(0, D_TISSUE), :], w1t, semt).start()
            w1_copy(0, 0).start()

        @pl.when(k + 1 < NK)
        def _():
            w1_copy(k + 1, (k + 1) % 2).start()

        @pl.when(k == 0)
        def _():
            pltpu.make_async_copy(w1_hbm.at[pl.ds(0, D_TISSUE), :], w1t, semt).wait()
            acc[...] = (
                jnp.dot(te_ref[...], w1t[...], preferred_element_type=jnp.float32)
                + b1_ref[...])

        w1_copy(k, k % 2).wait()
        part = jnp.dot(x_ref[0], w1buf[k % 2, pl.ds(0, 128), :],
                       preferred_element_type=jnp.float32)
        for qq in range(1, KB // 128):
            part += jnp.dot(x_ref[qq], w1buf[k % 2, pl.ds(128 * qq, 128), :],
                            preferred_element_type=jnp.float32)
        acc[...] += part

        @pl.when(k == NK - 1)
        def _():
            h = _gelu(acc[...])
            h2 = _gelu(jnp.dot(h, w2_ref[...], preferred_element_type=jnp.float32)
                       + b2_ref[...])
            y_ref[...] = (
                jnp.dot(h2, w3_ref[...], preferred_element_type=jnp.float32)
                + b3_ref[...])

    return pl.pallas_call(
        body,
        grid=(NK,),
        in_specs=[
            pl.BlockSpec((KB // 128, B, 128), lambda k: (k, 0, 0)),
            pl.BlockSpec((B, D_TISSUE), lambda k: (0, 0)),
            pl.BlockSpec((1, HIDDEN), lambda k: (0, 0)),
            pl.BlockSpec((HIDDEN, H2), lambda k: (0, 0)),
            pl.BlockSpec((1, H2), lambda k: (0, 0)),
            pl.BlockSpec((H2, 1), lambda k: (0, 0)),
            pl.BlockSpec((1, 1), lambda k: (0, 0)),
            pl.BlockSpec(memory_space=pl.ANY),
        ],
        out_specs=pl.BlockSpec((B, 1), lambda k: (0, 0)),
        out_shape=jax.ShapeDtypeStruct((B, 1), jnp.float32),
        scratch_shapes=[
            pltpu.VMEM((2, KB, HIDDEN), jnp.float32),
            pltpu.VMEM((D_TISSUE, HIDDEN), jnp.float32),
            pltpu.VMEM((B, HIDDEN), jnp.float32),
            pltpu.SemaphoreType.DMA((2,)),
            pltpu.SemaphoreType.DMA,
        ],
        compiler_params=pltpu.CompilerParams(
            dimension_semantics=("arbitrary",)),
    )(x2d, te, b1r, W2, b2r, W3, b3r, W1)


def kernel(rna_data, tissue_id, tissue_table, seq_table, W1, b1, W2, b2, W3, b3):
    pair_table = jnp.concatenate(
        [jnp.broadcast_to(seq_table[:, None, :], (VOCAB, VOCAB, D_TOKEN)),
         jnp.broadcast_to(seq_table[None, :, :], (VOCAB, VOCAB, D_TOKEN))],
        axis=2).reshape(VOCAB * VOCAB, D_PAIR)
    x3, te = _sc_gather(pair_table, rna_data, tissue_table, tissue_id)
    x6 = x3.reshape(K_SEQ // 128, B, 128)
    return _mlp(x6, te, b1.reshape(1, HIDDEN), W2, b2.reshape(1, H2),
                W3, b3.reshape(1, 1), W1)
